# baseline (speedup 1.0000x reference)
import functools
import math

import jax
import jax.numpy as jnp
from jax import lax
from jax.experimental import pallas as pl
from jax.experimental.pallas import tpu as pltpu
from jax.experimental.pallas import tpu_sc as plsc

_NW = 32
_G = 128
_VG = 7812


def _mesh():
    return plsc.VectorSubcoreMesh(
        core_axis_name="c", subcore_axis_name="s", num_cores=2, num_subcores=16
    )


@jax.jit
def _format_table(w_t, tail):
    scale = math.sqrt(64.0)

    @functools.partial(
        pl.kernel,
        out_type=jax.ShapeDtypeStruct((500032, 128), jnp.float32),
        mesh=_mesh(),
        scratch_types=[
            pltpu.VMEM((4, 64, 128), jnp.float32),
            pltpu.VMEM((4, 64, 128), jnp.float32),
            pltpu.VMEM((64, 64), jnp.float32),
            pltpu.SemaphoreType.DMA,
            pltpu.SemaphoreType.DMA,
            pltpu.SemaphoreType.DMA,
            pltpu.SemaphoreType.DMA,
            pltpu.SemaphoreType.DMA,
            pltpu.SemaphoreType.DMA,
            pltpu.SemaphoreType.DMA,
            pltpu.SemaphoreType.DMA,
        ],
        compiler_params=pltpu.CompilerParams(needs_layout_passes=False),
    )
    def fmt(
        wt_hbm, tail_hbm, tc_hbm, buf_in, buf_out, buf_tail,
        si0, si1, si2, si3, so0, so1, so2, so3,
    ):
        sem_i = (si0, si1, si2, si3)
        sem_o = (so0, so1, so2, so3)
        wid = lax.axis_index("s") * 2 + lax.axis_index("c")
        per = _VG // _NW
        rem = _VG - per * _NW
        start = wid * per + jnp.minimum(wid, rem)
        count = per + jnp.where(wid < rem, 1, 0)

        iota = lax.iota(jnp.int32, 16)

        def start_in(i, b):
            @pl.when(i < count)
            def _():
                pltpu.async_copy(
                    wt_hbm.at[:, pl.ds((start + i) * 128, 128)],
                    buf_in.at[b],
                    sem_i[b],
                )

        for b in range(4):
            start_in(jnp.int32(b), b)

        fvs = [k * 16 + iota for k in range(4)]

        def group(io, carry):
            for b in range(4):
                i = io * 4 + b

                @pl.when(i < count)
                def _do():
                    pltpu.make_async_copy(
                        wt_hbm.at[:, pl.ds(0, 128)], buf_in.at[b], sem_i[b]
                    ).wait()

                    @pl.when(i >= 4)
                    def _wo():
                        pltpu.make_async_copy(
                            buf_out.at[b], tc_hbm.at[pl.ds(0, 64)], sem_o[b]
                        ).wait()

                    def diag(j, carry2):
                        crot = (j + iota) & 15
                        ch = lax.shift_right_logical(crot, 1)
                        c2 = (crot & 1) * 64
                        cols = [c2 + fvs[k] for k in range(4)]
                        for cbk in range(8):
                            cv = cbk * 16 + crot
                            rowv = cbk * 8 + ch
                            for fb in range(4):
                                val = (
                                    plsc.load_gather(
                                        buf_in.at[b], [fvs[fb], cv]
                                    )
                                    * scale
                                )
                                plsc.store_scatter(
                                    buf_out.at[b], [rowv, cols[fb]], val
                                )
                        return carry2

                    lax.fori_loop(0, 16, diag, 0, unroll=4)
                    start_in(i + 4, b)
                    pltpu.async_copy(
                        buf_out.at[b],
                        tc_hbm.at[pl.ds((start + i) * 64, 64)],
                        sem_o[b],
                    )
            return carry

        lax.fori_loop(0, (count + 3) // 4 + 1, group, 0)

        for b in range(4):
            pltpu.make_async_copy(
                buf_out.at[b], tc_hbm.at[pl.ds(0, 64)], sem_o[b]
            ).wait()

        @pl.when(wid == 0)
        def _tail():
            pltpu.async_copy(tail_hbm, buf_tail, sem_i[0]).wait()

            def featt(v, carry2):
                row = jnp.full((16,), 0, jnp.int32) + lax.shift_right_logical(v, 1)
                for k in range(4):
                    f = k * 16 + iota
                    val = buf_tail[v, pl.ds(k * 16, 16)] * scale
                    col = (v & 1) * 64 + f
                    plsc.store_scatter(buf_out.at[0], [row, col], val)
                return carry2

            lax.fori_loop(0, 64, featt, 0, unroll=8)
            pltpu.sync_copy(
                buf_out.at[0, pl.ds(0, 32)], tc_hbm.at[pl.ds(499968, 32)]
            )

    return fmt(w_t, tail)


@jax.jit
def _gather(idx, table):
    d = 64
    n_streams_total, g_sz = idx.shape
    per_w = n_streams_total // _NW

    @functools.partial(
        pl.kernel,
        out_type=jax.ShapeDtypeStruct((200, 8, 32, 8, 128), jnp.float32),
        mesh=_mesh(),
        scratch_types=[
            pltpu.VMEM((per_w, _G), jnp.int32),
            pltpu.VMEM((6, _G, d), jnp.float32),
            pltpu.VMEM((2, 8, 8, _G + 1), jnp.float32),
            pltpu.SemaphoreType.DMA,
            pltpu.SemaphoreType.DMA,
            pltpu.SemaphoreType.DMA,
            pltpu.SemaphoreType.DMA,
            pltpu.SemaphoreType.DMA,
            pltpu.SemaphoreType.DMA,
            pltpu.SemaphoreType.DMA,
            pltpu.SemaphoreType.DMA,
        ],
        compiler_params=pltpu.CompilerParams(
            use_tc_tiling_on_sc=False, needs_layout_passes=False
        ),
    )
    def emb(
        idx_hbm, tab_hbm, out_hbm, idx_v, bufg, buft,
        sg0, sg1, sg2, sg3, sg4, sg5, so0, so1,
    ):
        wid = lax.axis_index("s") * 2 + lax.axis_index("c")
        base = wid * per_w
        pltpu.sync_copy(idx_hbm.at[pl.ds(base, per_w)], idx_v.at[...])
        sg = (sg0, sg1, sg2, sg3, sg4, sg5)
        so = (so0, so1)
        iota = lax.iota(jnp.int32, 16)

        def issue_gather(t, j):
            @pl.when(t < per_w)
            def _():
                pltpu.async_copy(tab_hbm.at[idx_v.at[t]], bufg.at[j], sg[j])

        def wait_gather(j):
            pltpu.make_async_copy(
                tab_hbm.at[pl.ds(0, _G)], bufg.at[j], sg[j]
            ).wait()

        def issue_out(t, p):
            st = base + t
            s = st // 32
            g = st - s * 32
            pltpu.async_copy(
                buft.at[p, :, :, pl.ds(0, _G)], out_hbm.at[s, :, g], so[p]
            )

        def wait_out(p):
            pltpu.make_async_copy(
                buft.at[p, :, :, pl.ds(0, _G)], out_hbm.at[0, :, 0], so[p]
            ).wait()

        for j in range(6):
            issue_gather(jnp.int32(j), j)

        def outer(io, carry):
            for j in range(6):
                t = io * 6 + j
                p = j % 2

                @pl.when(t < per_w)
                def _do():
                    wait_gather(j)

                    @pl.when(t >= 2)
                    def _wo():
                        wait_out(p)

                    def row(r, carry2):
                        rv = jnp.full((16,), 0, jnp.int32) + r
                        for k in range(d // 16):
                            f = k * 16 + iota
                            val = bufg[j, r, pl.ds(k * 16, 16)]
                            plsc.store_scatter(
                                buft.at[p],
                                [lax.shift_right_logical(f, 3), f & 7, rv],
                                val,
                            )
                        return carry2

                    lax.fori_loop(0, _G, row, 0, unroll=8)
                    issue_gather(t + 6, j)
                    issue_out(t, p)
            return carry

        lax.fori_loop(0, (per_w + 5) // 6, outer, 0)
        wait_out(0)
        wait_out(1)

    return emb(idx, table)


def kernel(tokens, embedding_weight):
    b, s = tokens.shape
    v, d = embedding_weight.shape
    idx = tokens.T.reshape(s * (b // _G), _G).astype(jnp.int32)
    w_t = embedding_weight.T
    tail = embedding_weight[_VG * 128 :, :]
    tc = _format_table(w_t, tail)
    tab = tc.reshape(1000064, 64)
    out5 = _gather(idx, tab)
    return out5.transpose(2, 4, 0, 1, 3).reshape(b, s, d)


# speedup vs baseline: 1.0731x; 1.0731x over previous
import functools
import math

import jax
import jax.numpy as jnp
from jax import lax
from jax.experimental import pallas as pl
from jax.experimental.pallas import tpu as pltpu
from jax.experimental.pallas import tpu_sc as plsc

_NW = 32
_G = 128
_VG = 7812


def _mesh():
    return plsc.VectorSubcoreMesh(
        core_axis_name="c", subcore_axis_name="s", num_cores=2, num_subcores=16
    )


@jax.jit
def _format_table(w_t, tail):
    scale = math.sqrt(64.0)

    @functools.partial(
        pl.kernel,
        out_type=jax.ShapeDtypeStruct((500032, 128), jnp.float32),
        mesh=_mesh(),
        scratch_types=[
            pltpu.VMEM((2, 64, 128), jnp.float32),
            pltpu.VMEM((2, 64, 128), jnp.float32),
            pltpu.VMEM((64, 64), jnp.float32),
            pltpu.SemaphoreType.DMA,
            pltpu.SemaphoreType.DMA,
            pltpu.SemaphoreType.DMA,
            pltpu.SemaphoreType.DMA,
        ],
        compiler_params=pltpu.CompilerParams(needs_layout_passes=False),
    )
    def fmt(
        wt_hbm, tail_hbm, tc_hbm, buf_in, buf_out, buf_tail, si0, si1, so0, so1
    ):
        sem_i = (si0, si1)
        sem_o = (so0, so1)
        wid = lax.axis_index("s") * 2 + lax.axis_index("c")
        per = _VG // _NW
        rem = _VG - per * _NW
        start = wid * per + jnp.minimum(wid, rem)
        count = per + jnp.where(wid < rem, 1, 0)

        iota = lax.iota(jnp.int32, 16)

        def start_in(i, b):
            @pl.when(i < count)
            def _():
                pltpu.async_copy(
                    wt_hbm.at[:, pl.ds((start + i) * 128, 128)],
                    buf_in.at[b],
                    sem_i[b],
                )

        for b in range(2):
            start_in(jnp.int32(b), b)

        fvs = [k * 16 + iota for k in range(4)]

        def group(io, carry):
            for b in range(2):
                i = io * 2 + b

                @pl.when(i < count)
                def _do():
                    pltpu.make_async_copy(
                        wt_hbm.at[:, pl.ds(0, 128)], buf_in.at[b], sem_i[b]
                    ).wait()

                    @pl.when(i >= 2)
                    def _wo():
                        pltpu.make_async_copy(
                            buf_out.at[b], tc_hbm.at[pl.ds(0, 64)], sem_o[b]
                        ).wait()

                    def diag(j, carry2):
                        crot = (j + iota) & 15
                        ch = lax.shift_right_logical(crot, 1)
                        c2 = (crot & 1) * 64
                        cols = [c2 + fvs[k] for k in range(4)]
                        for cbk in range(8):
                            cv = cbk * 16 + crot
                            rowv = cbk * 8 + ch
                            for fb in range(4):
                                val = (
                                    plsc.load_gather(
                                        buf_in.at[b], [fvs[fb], cv]
                                    )
                                    * scale
                                )
                                plsc.store_scatter(
                                    buf_out.at[b], [rowv, cols[fb]], val
                                )
                        return carry2

                    lax.fori_loop(0, 16, diag, 0, unroll=4)
                    start_in(i + 2, b)
                    pltpu.async_copy(
                        buf_out.at[b],
                        tc_hbm.at[pl.ds((start + i) * 64, 64)],
                        sem_o[b],
                    )
            return carry

        lax.fori_loop(0, (count + 1) // 2 + 1, group, 0)

        for b in range(2):
            pltpu.make_async_copy(
                buf_out.at[b], tc_hbm.at[pl.ds(0, 64)], sem_o[b]
            ).wait()

        @pl.when(wid == 0)
        def _tail():
            pltpu.async_copy(tail_hbm, buf_tail, sem_i[0]).wait()

            def featt(v, carry2):
                row = jnp.full((16,), 0, jnp.int32) + lax.shift_right_logical(v, 1)
                for k in range(4):
                    f = k * 16 + iota
                    val = buf_tail[v, pl.ds(k * 16, 16)] * scale
                    col = (v & 1) * 64 + f
                    plsc.store_scatter(buf_out.at[0], [row, col], val)
                return carry2

            lax.fori_loop(0, 64, featt, 0, unroll=8)
            pltpu.sync_copy(
                buf_out.at[0, pl.ds(0, 32)], tc_hbm.at[pl.ds(499968, 32)]
            )

    return fmt(w_t, tail)


@jax.jit
def _gather(idx, table):
    d = 64
    n_streams_total, g_sz = idx.shape
    per_w = n_streams_total // _NW

    @functools.partial(
        pl.kernel,
        out_type=jax.ShapeDtypeStruct((200, 8, 32, 8, 128), jnp.float32),
        mesh=_mesh(),
        scratch_types=[
            pltpu.VMEM((per_w, _G), jnp.int32),
            pltpu.VMEM((4, _G, d), jnp.float32),
            pltpu.VMEM((2, 8, 8, _G + 1), jnp.float32),
            pltpu.SemaphoreType.DMA,
            pltpu.SemaphoreType.DMA,
            pltpu.SemaphoreType.DMA,
            pltpu.SemaphoreType.DMA,
            pltpu.SemaphoreType.DMA,
            pltpu.SemaphoreType.DMA,
        ],
        compiler_params=pltpu.CompilerParams(
            use_tc_tiling_on_sc=False, needs_layout_passes=False
        ),
    )
    def emb(
        idx_hbm, tab_hbm, out_hbm, idx_v, bufg, buft,
        sg0, sg1, sg2, sg3, so0, so1,
    ):
        wid = lax.axis_index("s") * 2 + lax.axis_index("c")
        base = wid * per_w
        pltpu.sync_copy(idx_hbm.at[pl.ds(base, per_w)], idx_v.at[...])
        sg = (sg0, sg1, sg2, sg3)
        so = (so0, so1)
        iota = lax.iota(jnp.int32, 16)

        def issue_gather(t, j):
            @pl.when(t < per_w)
            def _():
                pltpu.async_copy(tab_hbm.at[idx_v.at[t]], bufg.at[j], sg[j])

        def wait_gather(j):
            pltpu.make_async_copy(
                tab_hbm.at[pl.ds(0, _G)], bufg.at[j], sg[j]
            ).wait()

        def issue_out(t, p):
            st = base + t
            s = st // 32
            g = st - s * 32
            pltpu.async_copy(
                buft.at[p, :, :, pl.ds(0, _G)], out_hbm.at[s, :, g], so[p]
            )

        def wait_out(p):
            pltpu.make_async_copy(
                buft.at[p, :, :, pl.ds(0, _G)], out_hbm.at[0, :, 0], so[p]
            ).wait()

        for j in range(4):
            issue_gather(jnp.int32(j), j)

        def outer(io, carry):
            for j in range(4):
                t = io * 4 + j
                p = j % 2

                @pl.when(t < per_w)
                def _do():
                    wait_gather(j)

                    @pl.when(t >= 2)
                    def _wo():
                        wait_out(p)

                    def row(r, carry2):
                        rv = jnp.full((16,), 0, jnp.int32) + r
                        for k in range(d // 16):
                            f = k * 16 + iota
                            val = bufg[j, r, pl.ds(k * 16, 16)]
                            plsc.store_scatter(
                                buft.at[p],
                                [lax.shift_right_logical(f, 3), f & 7, rv],
                                val,
                            )
                        return carry2

                    lax.fori_loop(0, _G, row, 0, unroll=8)
                    issue_gather(t + 4, j)
                    issue_out(t, p)
            return carry

        lax.fori_loop(0, (per_w + 3) // 4, outer, 0)
        wait_out(0)
        wait_out(1)

    return emb(idx, table)


def kernel(tokens, embedding_weight):
    b, s = tokens.shape
    v, d = embedding_weight.shape
    idx = tokens.T.reshape(s * (b // _G), _G).astype(jnp.int32)
    w_t = embedding_weight.T
    tail = embedding_weight[_VG * 128 :, :]
    tc = _format_table(w_t, tail)
    tab = tc.reshape(1000064, 64)
    out5 = _gather(idx, tab)
    return out5.transpose(2, 4, 0, 1, 3).reshape(b, s, d)
